# baseline (speedup 1.0000x reference)
import jax
import jax.numpy as jnp
from jax.experimental import pallas as pl
from jax.experimental.pallas import tpu as pltpu

B = 16384
D = 3072
NUM_CLASSES = 10
RB = 1024
NBLK = B // RB
PAIR = 2


def _body(x_ref, y_ref, xout_ref, oh_ref):
    i = pl.program_id(0)
    xout_ref[...] = x_ref[...]
    yv = y_ref[pl.ds(i * RB, RB)]
    y2 = yv.reshape(RB, 1)
    iota = jax.lax.broadcasted_iota(jnp.int32, (RB, NUM_CLASSES), 1)
    oh_ref[pl.ds((i % PAIR) * RB, RB), :] = (y2 == iota).astype(jnp.float32)


def kernel(x, y):
    x_out, one_hot = pl.pallas_call(
        _body,
        grid=(NBLK,),
        in_specs=[
            pl.BlockSpec((RB, D), lambda i: (i, 0)),
            pl.BlockSpec((B,), lambda i: (0,)),
        ],
        out_specs=[
            pl.BlockSpec((RB, D), lambda i: (i, 0)),
            pl.BlockSpec((RB * PAIR, NUM_CLASSES), lambda i: (i // PAIR, 0)),
        ],
        out_shape=[
            jax.ShapeDtypeStruct((B, D), jnp.float32),
            jax.ShapeDtypeStruct((B, NUM_CLASSES), jnp.float32),
        ],
        compiler_params=pltpu.CompilerParams(
            dimension_semantics=("arbitrary",),
        ),
    )(x, y)
    return (x_out, one_hot)


# speedup vs baseline: 1.0009x; 1.0009x over previous
import jax
import jax.numpy as jnp
from jax.experimental import pallas as pl
from jax.experimental.pallas import tpu as pltpu

B = 16384
D = 3072
NUM_CLASSES = 10
RB = 1024
NBLK = B // RB


def _body(x_ref, y_ref, xout_ref, oh_ref):
    i = pl.program_id(0)
    xout_ref[...] = x_ref[...]
    yv = y_ref[pl.ds(i * RB, RB)]
    y2 = yv.reshape(RB, 1)
    iota = jax.lax.broadcasted_iota(jnp.int32, (RB, NUM_CLASSES), 1)
    oh_ref[...] = (y2 == iota).astype(jnp.float32)


def kernel(x, y):
    x_out, one_hot = pl.pallas_call(
        _body,
        grid=(NBLK,),
        in_specs=[
            pl.BlockSpec((RB, D), lambda i: (i, 0)),
            pl.BlockSpec((B,), lambda i: (0,)),
        ],
        out_specs=[
            pl.BlockSpec((RB, D), lambda i: (i, 0)),
            pl.BlockSpec((RB, NUM_CLASSES), lambda i: (i, 0)),
        ],
        out_shape=[
            jax.ShapeDtypeStruct((B, D), jnp.float32),
            jax.ShapeDtypeStruct((B, NUM_CLASSES), jnp.float32),
        ],
        compiler_params=pltpu.CompilerParams(
            dimension_semantics=("arbitrary",),
        ),
    )(x, y)
    return (x_out, one_hot)
